# baseline (speedup 1.0000x reference)
import dataclasses

import jax
import jax.numpy as jnp
from jax import lax
from jax.experimental import pallas as pl
from jax.experimental.pallas import tpu as pltpu
from jax.experimental.pallas import tpu_sc as plsc

N = 320000
D = 128
G = 1024
NC = 2
NS = 16
NW = NC * NS
RPW = N // NW
CH = 200
HALF = CH // 2
NCHUNKS = RPW // CH
GPS = G // NS
NV = D // 16


NBUF = 4
FAHEAD = 2


def _sc_body(y_hbm, b3_hbm, out_hbm, idx_v, *rest):
    bufs = rest[:NBUF]
    flbuf, flidx, acc_s, st_s = rest[NBUF:NBUF + 4]
    sems = rest[NBUF + 4:NBUF + 4 + NBUF]
    sscs = rest[NBUF + 4 + NBUF:]
    cid = lax.axis_index("c")
    sid = lax.axis_index("s")
    wid = sid * NC + cid
    base = wid * RPW

    zeros16 = jnp.zeros((16,), jnp.float32)

    pltpu.sync_copy(b3_hbm.at[wid], idx_v)

    @pl.loop(0, 16)
    def _(i):
        for k in range(NV):
            flbuf[i, pl.ds(16 * k, 16)] = zeros16

    @pl.loop(0, GPS)
    def _(i):
        for k in range(NV):
            bufs[0][i, pl.ds(16 * k, 16)] = zeros16

    pltpu.sync_copy(bufs[0].at[pl.ds(0, GPS)],
                    acc_s.at[pl.ds(sid * GPS, GPS)])
    st_s[0] = 0
    st_s[1] = 0
    for b in range(NBUF):
        st_s[2 + b] = 0
    plsc.subcore_barrier()

    def start_fetch(j, buf, sem):
        pltpu.async_copy(y_hbm.at[pl.ds(base + j * CH, CH)], buf, sem)

    def wait_fetch(j, buf, sem):
        pltpu.make_async_copy(y_hbm.at[pl.ds(base + j * CH, CH)], buf,
                              sem).wait()

    def flush(cur):
        flidx[0] = jnp.full((16,), cur, jnp.int32)
        pltpu.sync_copy(flbuf, acc_s.at[flidx.at[0]], add=True)
        for k in range(NV):
            flbuf[0, pl.ds(16 * k, 16)] = zeros16

    def drain(b):
        @pl.when(st_s[2 + b] != 0)
        def _():
            for h in range(2):
                pltpu.make_async_copy(bufs[b].at[pl.ds(h * HALF, HALF)],
                                      acc_s.at[idx_v.at[0, h]],
                                      sscs[b]).wait()
            st_s[2 + b] = 0

    def process(j, buf, b):
        id0 = jnp.min(idx_v[j, 0, pl.ds(0, 16)])
        idl = jnp.max(idx_v[j, 1, pl.ds(HALF - 16, 16)])
        cur = st_s[0]
        dirty = st_s[1]
        mixed = id0 != idl

        @pl.when((dirty != 0) & ((cur != id0) | mixed))
        def _():
            flush(cur)
            st_s[1] = 0

        @pl.when(mixed)
        def _():
            for h in range(2):
                pltpu.async_copy(buf.at[pl.ds(h * HALF, HALF)],
                                 acc_s.at[idx_v.at[j, h]], sscs[b],
                                 add=True)
            st_s[0] = idl
            st_s[1] = 0
            st_s[2 + b] = 1

        @pl.when(jnp.logical_not(mixed))
        def _():
            def row_body(r, acc):
                return tuple(acc[k] + buf[r, pl.ds(16 * k, 16)]
                             for k in range(NV))

            acc = lax.fori_loop(0, CH, row_body,
                                tuple(jnp.zeros((16,), jnp.float32)
                                      for _ in range(NV)),
                                unroll=8)
            for k in range(NV):
                sl = pl.ds(16 * k, 16)
                flbuf[0, sl] += acc[k]

            st_s[0] = id0
            st_s[1] = 1

    for j in range(FAHEAD):
        start_fetch(j, bufs[j], sems[j])

    @pl.loop(0, (NCHUNKS - FAHEAD) // NBUF)
    def _(k):
        j0 = NBUF * k
        for b in range(NBUF):
            j = j0 + b
            wait_fetch(j, bufs[b], sems[b])
            process(j, bufs[b], b)
            bf = (b + FAHEAD) % NBUF
            drain(bf)
            start_fetch(j + FAHEAD, bufs[bf], sems[bf])

    for j in range(NCHUNKS - FAHEAD, NCHUNKS):
        b = j % NBUF
        wait_fetch(j, bufs[b], sems[b])
        process(j, bufs[b], b)

    for b in range(NBUF):
        drain(b)

    @pl.when(st_s[1] != 0)
    def _():
        flush(st_s[0])

    plsc.subcore_barrier()
    pltpu.sync_copy(acc_s.at[pl.ds(sid * GPS, GPS)],
                    out_hbm.at[cid, pl.ds(sid * GPS, GPS)])


def _sum_body(p_ref, o_ref):
    o_ref[...] = p_ref[0] + p_ref[1]


def kernel(y, batch):
    b3 = batch.reshape(NW, NCHUNKS, 2, HALF)
    mesh = plsc.VectorSubcoreMesh(core_axis_name="c", subcore_axis_name="s")
    cp = pltpu.CompilerParams()
    if "needs_layout_passes" in pltpu.CompilerParams.__dataclass_fields__:
        cp = dataclasses.replace(cp, needs_layout_passes=False)
    sc_call = pl.kernel(
        _sc_body,
        out_type=jax.ShapeDtypeStruct((NC, G, D), jnp.float32),
        mesh=mesh,
        compiler_params=cp,
        scratch_types=(
            [pltpu.VMEM((NCHUNKS, 2, HALF), jnp.int32)]
            + [pltpu.VMEM((CH, D), jnp.float32) for _ in range(NBUF)]
            + [pltpu.VMEM((16, D), jnp.float32),
               pltpu.VMEM((2, 16), jnp.int32),
               pltpu.VMEM_SHARED((G, D), jnp.float32),
               pltpu.SMEM((2 + NBUF,), jnp.int32)]
            + [pltpu.SemaphoreType.DMA for _ in range(2 * NBUF)]
        ),
    )
    partials = sc_call(y, b3)
    return pl.pallas_call(
        _sum_body,
        out_shape=jax.ShapeDtypeStruct((G, D), jnp.float32),
    )(partials)


# speedup vs baseline: 1.3776x; 1.3776x over previous
import dataclasses

import jax
import jax.numpy as jnp
from jax import lax
from jax.experimental import pallas as pl
from jax.experimental.pallas import tpu as pltpu
from jax.experimental.pallas import tpu_sc as plsc

N = 320000
D = 128
G = 1024
NC = 2
NS = 16
NW = NC * NS
RPW = N // NW
CH = 80
NCHUNKS = RPW // CH
GPS = G // NS
NV = D // 16


NBUF = 8
FAHEAD = 5


def _sc_body(y_hbm, b3_hbm, out_hbm, idx_v, *rest):
    bufs = rest[:NBUF]
    flbuf, flidx, acc_s, st_s = rest[NBUF:NBUF + 4]
    sems = rest[NBUF + 4:NBUF + 4 + NBUF]
    sscs = rest[NBUF + 4 + NBUF:]
    cid = lax.axis_index("c")
    sid = lax.axis_index("s")
    wid = sid * NC + cid
    base = wid * RPW

    zeros16 = jnp.zeros((16,), jnp.float32)

    pltpu.sync_copy(b3_hbm.at[wid], idx_v)

    @pl.loop(0, 16)
    def _(i):
        for k in range(NV):
            flbuf[i, pl.ds(16 * k, 16)] = zeros16

    @pl.loop(0, GPS)
    def _(i):
        for k in range(NV):
            bufs[0][i, pl.ds(16 * k, 16)] = zeros16

    pltpu.sync_copy(bufs[0].at[pl.ds(0, GPS)],
                    acc_s.at[pl.ds(sid * GPS, GPS)])
    st_s[0] = 0
    st_s[1] = 0
    for b in range(NBUF):
        st_s[2 + b] = 0
    plsc.subcore_barrier()

    def start_fetch(j, buf, sem):
        pltpu.async_copy(y_hbm.at[pl.ds(base + j * CH, CH)], buf, sem)

    def wait_fetch(j, buf, sem):
        pltpu.make_async_copy(y_hbm.at[pl.ds(base + j * CH, CH)], buf,
                              sem).wait()

    def flush(cur):
        flidx[0] = jnp.full((16,), cur, jnp.int32)
        pltpu.sync_copy(flbuf, acc_s.at[flidx.at[0]], add=True)
        for k in range(NV):
            flbuf[0, pl.ds(16 * k, 16)] = zeros16

    def drain(b):
        @pl.when(st_s[2 + b] != 0)
        def _():
            pltpu.make_async_copy(bufs[b], acc_s.at[idx_v.at[0]],
                                  sscs[b]).wait()
            st_s[2 + b] = 0

    def process(j, buf, b):
        vlo = idx_v[j, pl.ds(0, 16)]
        vhi = vlo
        for t in range(1, CH // 16):
            v = idx_v[j, pl.ds(16 * t, 16)]
            vlo = jnp.minimum(vlo, v)
            vhi = jnp.maximum(vhi, v)
        id0 = jnp.min(vlo)
        idl = jnp.max(vhi)
        cur = st_s[0]
        dirty = st_s[1]
        mixed = id0 != idl

        @pl.when((dirty != 0) & ((cur != id0) | mixed))
        def _():
            flush(cur)
            st_s[1] = 0

        @pl.when(mixed)
        def _():
            pltpu.async_copy(buf, acc_s.at[idx_v.at[j]], sscs[b], add=True)
            st_s[0] = idl
            st_s[1] = 0
            st_s[2 + b] = 1

        @pl.when(jnp.logical_not(mixed))
        def _():
            def row_body(r, acc):
                return tuple(acc[k] + buf[r, pl.ds(16 * k, 16)]
                             for k in range(NV))

            acc = lax.fori_loop(0, CH, row_body,
                                tuple(jnp.zeros((16,), jnp.float32)
                                      for _ in range(NV)),
                                unroll=8)
            for k in range(NV):
                sl = pl.ds(16 * k, 16)
                flbuf[0, sl] += acc[k]

            st_s[0] = id0
            st_s[1] = 1

    for j in range(FAHEAD):
        start_fetch(j, bufs[j], sems[j])

    @pl.loop(0, (NCHUNKS - FAHEAD) // NBUF)
    def _(k):
        j0 = NBUF * k
        for b in range(NBUF):
            j = j0 + b
            wait_fetch(j, bufs[b], sems[b])
            process(j, bufs[b], b)
            bf = (b + FAHEAD) % NBUF
            drain(bf)
            start_fetch(j + FAHEAD, bufs[bf], sems[bf])

    for j in range(NCHUNKS - FAHEAD, NCHUNKS):
        b = j % NBUF
        wait_fetch(j, bufs[b], sems[b])
        process(j, bufs[b], b)

    for b in range(NBUF):
        drain(b)

    @pl.when(st_s[1] != 0)
    def _():
        flush(st_s[0])

    plsc.subcore_barrier()
    pltpu.sync_copy(acc_s.at[pl.ds(sid * GPS, GPS)],
                    out_hbm.at[cid, pl.ds(sid * GPS, GPS)])


def _sum_body(p_ref, o_ref):
    o_ref[...] = p_ref[0] + p_ref[1]


def kernel(y, batch):
    b3 = batch.reshape(NW, NCHUNKS, CH)
    mesh = plsc.VectorSubcoreMesh(core_axis_name="c", subcore_axis_name="s")
    cp = pltpu.CompilerParams()
    if "needs_layout_passes" in pltpu.CompilerParams.__dataclass_fields__:
        cp = dataclasses.replace(cp, needs_layout_passes=False)
    sc_call = pl.kernel(
        _sc_body,
        out_type=jax.ShapeDtypeStruct((NC, G, D), jnp.float32),
        mesh=mesh,
        compiler_params=cp,
        scratch_types=(
            [pltpu.VMEM((NCHUNKS, CH), jnp.int32)]
            + [pltpu.VMEM((CH, D), jnp.float32) for _ in range(NBUF)]
            + [pltpu.VMEM((16, D), jnp.float32),
               pltpu.VMEM((2, 16), jnp.int32),
               pltpu.VMEM_SHARED((G, D), jnp.float32),
               pltpu.SMEM((2 + NBUF,), jnp.int32)]
            + [pltpu.SemaphoreType.DMA for _ in range(2 * NBUF)]
        ),
    )
    partials = sc_call(y, b3)
    return pl.pallas_call(
        _sum_body,
        out_shape=jax.ShapeDtypeStruct((G, D), jnp.float32),
    )(partials)
